# baseline (speedup 1.0000x reference)
import functools

import jax
import jax.numpy as jnp
import numpy as np
from jax import lax
from jax.experimental import pallas as pl
from jax.experimental.pallas import tpu as pltpu
from jax.experimental.pallas import tpu_sc as plsc

_CLS_NUM_LIST = [5000 // (i + 1) for i in range(100)]
_MAX_M = 0.5
_S = 30.0

_B = 16384
_C = 100
_NW = 32
_RPW = _B // _NW
_NG = _RPW // 16

_LN2 = 0.6931471805599453


def _poly_log(s):
    bits = plsc.bitcast(s, jnp.int32)
    e = ((bits >> 23) & 255) - 127
    mant = plsc.bitcast((bits & 0x7FFFFF) | 0x3F800000, jnp.float32)
    t = (mant - 1.0) / (mant + 1.0)
    t2 = t * t
    p = jnp.float32(1.0 / 9.0)
    for c in (1.0 / 7.0, 1.0 / 5.0, 1.0 / 3.0, 1.0):
        p = p * t2 + jnp.float32(c)
    return e.astype(jnp.float32) * jnp.float32(_LN2) + (2.0 * t) * p


def _make_sc_kernel():
    mesh = plsc.VectorSubcoreMesh(core_axis_name="c", subcore_axis_name="s")

    @functools.partial(
        pl.kernel,
        mesh=mesh,
        compiler_params=pltpu.CompilerParams(needs_layout_passes=False),
        out_type=jax.ShapeDtypeStruct((_NW, 16), jnp.float32),
        scratch_types=[
            pltpu.VMEM((_RPW * _C,), jnp.float32),
            pltpu.VMEM((_RPW,), jnp.int32),
            pltpu.VMEM((_RPW,), jnp.float32),
            pltpu.VMEM((128,), jnp.float32),
            pltpu.VMEM((16,), jnp.float32),
        ],
    )
    def k(x_hbm, t_hbm, w_hbm, m_hbm, out_hbm, x_v, t_v, w_v, m_v, acc_v):
        wid = lax.axis_index("s") * 2 + lax.axis_index("c")
        row0 = wid * _RPW
        pltpu.sync_copy(x_hbm.at[pl.ds(row0 * _C, _RPW * _C)], x_v)
        pltpu.sync_copy(t_hbm.at[pl.ds(row0, _RPW)], t_v)
        pltpu.sync_copy(w_hbm.at[pl.ds(row0, _RPW)], w_v)
        pltpu.sync_copy(m_hbm, m_v)

        lane = lax.iota(jnp.int32, 16)
        ninf = jnp.full((16,), -3.0e38, jnp.float32)
        zero = jnp.zeros((16,), jnp.float32)

        def group(g, acc):
            rows = g * 16 + lane
            base = rows * _C
            tvec = plsc.load_gather(t_v, [rows])
            wvec = plsc.load_gather(w_v, [rows])
            mt = plsc.load_gather(m_v, [tvec])
            tidx = base + tvec
            xt = plsc.load_gather(x_v, [tidx])
            xt_m = xt - mt
            plsc.store_scatter(x_v, [tidx], xt_m)

            mx = [ninf, ninf, ninf, ninf]
            for j in range(_C):
                v = plsc.load_gather(x_v, [base + j])
                mx[j % 4] = jnp.maximum(mx[j % 4], v)
            m01 = jnp.maximum(mx[0], mx[1])
            m23 = jnp.maximum(mx[2], mx[3])
            big_m = jnp.float32(_S) * jnp.maximum(m01, m23)

            sa = [zero, zero, zero, zero]
            for j in range(_C):
                v = plsc.load_gather(x_v, [base + j])
                sa[j % 4] = sa[j % 4] + jnp.exp(jnp.float32(_S) * v - big_m)
            s = (sa[0] + sa[1]) + (sa[2] + sa[3])

            ce = _poly_log(s) + big_m - jnp.float32(_S) * xt_m
            return acc + ce * wvec

        acc = lax.fori_loop(0, _NG, group, zero)
        acc_v[...] = acc
        pltpu.sync_copy(acc_v, out_hbm.at[wid])

    return k


def kernel(x, target, instance_weights):
    assert x.shape == (_B, _C) and x.dtype == jnp.float32
    m_np = 1.0 / np.sqrt(np.sqrt(np.array(_CLS_NUM_LIST, dtype=np.float64)))
    m_np = m_np * (_MAX_M / np.max(m_np))
    m_pad = np.zeros(128, np.float32)
    m_pad[:_C] = m_np
    m_list = jnp.asarray(m_pad)

    partials = _make_sc_kernel()(
        x.reshape(-1),
        target.astype(jnp.int32),
        instance_weights,
        m_list,
    )
    return jnp.sum(partials) * jnp.float32(1.0 / _B)
